# baseline (speedup 1.0000x reference)
import functools

import jax
import jax.numpy as jnp
from jax import lax
from jax.experimental import pallas as pl
from jax.experimental.pallas import tpu as pltpu
from jax.experimental.pallas import tpu_sc as plsc

BATCH = 16384
EMB = 64
HID = 256

_NC = 2
_NS = 16
_NW = _NC * _NS
_ROWS_PER_W = BATCH // _NW
_IDX_CHUNK = 128
_NCHUNK = _ROWS_PER_W // _IDX_CHUNK


def _gather_body(uid_hbm, iid_hbm, uemb_hbm, iemb_hbm, uout_hbm, iout_hbm,
                 idx_u, idx_i, rows_u, rows_i, sem):
    wid = lax.axis_index("s") * _NC + lax.axis_index("c")
    r0 = wid * _NCHUNK
    pltpu.sync_copy(uid_hbm.at[pl.ds(r0, _NCHUNK)], idx_u)
    pltpu.sync_copy(iid_hbm.at[pl.ds(r0, _NCHUNK)], idx_i)
    copies = []
    for j in range(_NCHUNK):
        dst = pl.ds(j * _IDX_CHUNK, _IDX_CHUNK)
        copies.append(pltpu.async_copy(uemb_hbm.at[idx_u.at[j]], rows_u.at[dst], sem))
        copies.append(pltpu.async_copy(iemb_hbm.at[idx_i.at[j]], rows_i.at[dst], sem))
    for c in copies:
        c.wait()
    base = wid * _ROWS_PER_W
    pltpu.sync_copy(rows_u, uout_hbm.at[pl.ds(base, _ROWS_PER_W)])
    pltpu.sync_copy(rows_i, iout_hbm.at[pl.ds(base, _ROWS_PER_W)])


def _sc_gather(uid2d, iid2d, user_emb, item_emb):
    mesh = plsc.VectorSubcoreMesh(core_axis_name="c", subcore_axis_name="s")
    out_type = (
        jax.ShapeDtypeStruct((BATCH, EMB), jnp.float32),
        jax.ShapeDtypeStruct((BATCH, EMB), jnp.float32),
    )
    scratch = [
        pltpu.VMEM((_NCHUNK, _IDX_CHUNK), jnp.int32),
        pltpu.VMEM((_NCHUNK, _IDX_CHUNK), jnp.int32),
        pltpu.VMEM((_ROWS_PER_W, EMB), jnp.float32),
        pltpu.VMEM((_ROWS_PER_W, EMB), jnp.float32),
        pltpu.SemaphoreType.DMA,
    ]
    return pl.kernel(
        _gather_body, mesh=mesh, out_type=out_type, scratch_types=scratch,
        compiler_params=pltpu.CompilerParams(use_tc_tiling_on_sc=False),
        name="ncf_sc_gather",
    )(uid2d, iid2d, user_emb, item_emb)


_BLK = 2048


def _mlp_body(u_ref, i_ref, w1u_ref, w1i_ref, b1_ref, w2_ref, b2_ref, o_ref):
    h = (jnp.dot(u_ref[...], w1u_ref[...], preferred_element_type=jnp.float32)
         + jnp.dot(i_ref[...], w1i_ref[...], preferred_element_type=jnp.float32)
         + b1_ref[...])
    h = jnp.maximum(h, 0.0)
    s = jnp.sum(h * w2_ref[...], axis=1, keepdims=True) + b2_ref[...]
    o_ref[...] = 1.0 / (1.0 + jnp.exp(-s))


def _tc_mlp(uvec, ivec, w1u, w1i, b1r, w2r, b2r):
    grid = (BATCH // _BLK,)
    return pl.pallas_call(
        _mlp_body,
        grid=grid,
        in_specs=[
            pl.BlockSpec((_BLK, EMB), lambda i: (i, 0)),
            pl.BlockSpec((_BLK, EMB), lambda i: (i, 0)),
            pl.BlockSpec((EMB, HID), lambda i: (0, 0)),
            pl.BlockSpec((EMB, HID), lambda i: (0, 0)),
            pl.BlockSpec((1, HID), lambda i: (0, 0)),
            pl.BlockSpec((1, HID), lambda i: (0, 0)),
            pl.BlockSpec((1, 1), lambda i: (0, 0)),
        ],
        out_specs=pl.BlockSpec((_BLK, 1), lambda i: (i, 0)),
        out_shape=jax.ShapeDtypeStruct((BATCH, 1), jnp.float32),
    )(uvec, ivec, w1u, w1i, b1r, w2r, b2r)


def kernel(user_id, item_id, user_emb, item_emb, W1, b1, W2, b2):
    uid2d = user_id.astype(jnp.int32).reshape(BATCH // _IDX_CHUNK, _IDX_CHUNK)
    iid2d = item_id.astype(jnp.int32).reshape(BATCH // _IDX_CHUNK, _IDX_CHUNK)
    uvec, ivec = _sc_gather(uid2d, iid2d, user_emb, item_emb)
    w1u = W1[:EMB]
    w1i = W1[EMB:]
    b1r = b1.reshape(1, HID)
    w2r = W2.reshape(1, HID)
    b2r = b2.reshape(1, 1)
    return _tc_mlp(uvec, ivec, w1u, w1i, b1r, w2r, b2r)
